# baseline (speedup 1.0000x reference)
import jax
import jax.numpy as jnp
from jax.experimental import pallas as pl
from jax.experimental.pallas import tpu as pltpu

TOKENS = 32768
HIDDEN = 4096
N_EXPERTS = 64
TOP_K = 8
TILE = 2048
KSPLIT = 2
KCHUNK = HIDDEN // KSPLIT


def _gate_kernel(x_ref, w_ref, probs_ref, vals_ref, idx_ref, acc_ref):
    k = pl.program_id(1)
    partial = jax.lax.dot_general(
        x_ref[...],
        w_ref[...],
        (((1,), (1,)), ((), ())),
        preferred_element_type=jnp.float32,
    )

    @pl.when(k == 0)
    def _():
        acc_ref[...] = partial

    @pl.when(k == KSPLIT - 1)
    def _():
        logits = acc_ref[...] + partial
        m = jnp.max(logits, axis=-1, keepdims=True)
        e = jnp.exp(logits - m)
        s = jnp.sum(e, axis=-1, keepdims=True)
        probs = e / s
        probs_ref[...] = probs

        work = probs
        iota = jax.lax.broadcasted_iota(jnp.int32, probs.shape, 1).astype(
            jnp.float32
        )
        vals = []
        idxs = []
        for _ in range(TOP_K):
            v = jnp.max(work, axis=-1, keepdims=True)
            i = jnp.min(
                jnp.where(work == v, iota, float(N_EXPERTS)),
                axis=-1,
                keepdims=True,
            )
            vals.append(v)
            idxs.append(i)
            work = jnp.where(iota == i, -jnp.inf, work)
        top_vals = jnp.concatenate(vals, axis=-1)
        top_idx = jnp.concatenate(idxs, axis=-1)
        top_vals = top_vals / jnp.sum(top_vals, axis=-1, keepdims=True)
        vals_ref[...] = top_vals
        idx_ref[...] = top_idx.astype(jnp.int32)


@jax.jit
def kernel(x, W):
    n_tiles = TOKENS // TILE
    probs, top_vals, top_idx = pl.pallas_call(
        _gate_kernel,
        grid=(n_tiles, KSPLIT),
        in_specs=[
            pl.BlockSpec((TILE, KCHUNK), lambda i, k: (i, k)),
            pl.BlockSpec((N_EXPERTS, KCHUNK), lambda i, k: (0, k)),
        ],
        out_specs=[
            pl.BlockSpec((TILE, N_EXPERTS), lambda i, k: (i, 0)),
            pl.BlockSpec((TILE, TOP_K), lambda i, k: (i, 0)),
            pl.BlockSpec((TILE, TOP_K), lambda i, k: (i, 0)),
        ],
        out_shape=[
            jax.ShapeDtypeStruct((TOKENS, N_EXPERTS), jnp.float32),
            jax.ShapeDtypeStruct((TOKENS, TOP_K), jnp.float32),
            jax.ShapeDtypeStruct((TOKENS, TOP_K), jnp.int32),
        ],
        scratch_shapes=[pltpu.VMEM((TILE, N_EXPERTS), jnp.float32)],
        compiler_params=pltpu.CompilerParams(
            dimension_semantics=("parallel", "arbitrary"),
        ),
    )(x, W)
    return (probs, top_vals, top_idx)


# speedup vs baseline: 1.2423x; 1.2423x over previous
import jax
import jax.numpy as jnp
from jax.experimental import pallas as pl
from jax.experimental.pallas import tpu as pltpu

TOKENS = 32768
HIDDEN = 4096
N_EXPERTS = 64
TOP_K = 8
TILE = 1024


def _gate_kernel(x_ref, w_ref, probs_ref, vals_ref, idx_ref):
    x = x_ref[...]
    w = w_ref[...]
    logits = jax.lax.dot_general(
        x, w, (((1,), (1,)), ((), ())), preferred_element_type=jnp.float32
    )
    m = jnp.max(logits, axis=-1, keepdims=True)
    e = jnp.exp(logits - m)
    s = jnp.sum(e, axis=-1, keepdims=True)
    probs = e / s
    probs_ref[...] = probs

    work = probs
    iota = jax.lax.broadcasted_iota(jnp.int32, probs.shape, 1).astype(
        jnp.float32
    )
    vals = []
    idxs = []
    for _ in range(TOP_K):
        v = jnp.max(work, axis=-1, keepdims=True)
        i = jnp.min(
            jnp.where(work == v, iota, float(N_EXPERTS)),
            axis=-1,
            keepdims=True,
        )
        vals.append(v)
        idxs.append(i)
        work = jnp.where(iota == i, -jnp.inf, work)
    top_vals = jnp.concatenate(vals, axis=-1)
    top_idx = jnp.concatenate(idxs, axis=-1)
    top_vals = top_vals / jnp.sum(top_vals, axis=-1, keepdims=True)
    vals_ref[...] = top_vals
    idx_ref[...] = top_idx.astype(jnp.int32)


@jax.jit
def kernel(x, W):
    n_tiles = TOKENS // TILE
    probs, top_vals, top_idx = pl.pallas_call(
        _gate_kernel,
        grid=(n_tiles,),
        in_specs=[
            pl.BlockSpec((TILE, HIDDEN), lambda i: (i, 0)),
            pl.BlockSpec((N_EXPERTS, HIDDEN), lambda i: (0, 0)),
        ],
        out_specs=[
            pl.BlockSpec((TILE, N_EXPERTS), lambda i: (i, 0)),
            pl.BlockSpec((TILE, TOP_K), lambda i: (i, 0)),
            pl.BlockSpec((TILE, TOP_K), lambda i: (i, 0)),
        ],
        out_shape=[
            jax.ShapeDtypeStruct((TOKENS, N_EXPERTS), jnp.float32),
            jax.ShapeDtypeStruct((TOKENS, TOP_K), jnp.float32),
            jax.ShapeDtypeStruct((TOKENS, TOP_K), jnp.int32),
        ],
        compiler_params=pltpu.CompilerParams(
            dimension_semantics=("arbitrary",),
        ),
    )(x, W)
    return (probs, top_vals, top_idx)


# speedup vs baseline: 1.2451x; 1.0023x over previous
import jax
import jax.numpy as jnp
from jax.experimental import pallas as pl
from jax.experimental.pallas import tpu as pltpu

TOKENS = 32768
HIDDEN = 4096
N_EXPERTS = 64
TOP_K = 8
TILE = 1024


def _gate_kernel(x_ref, w_ref, probs_ref, vals_ref, idx_ref):
    x = x_ref[...]
    w = w_ref[...]
    logits = jax.lax.dot_general(
        x, w, (((1,), (1,)), ((), ())), preferred_element_type=jnp.float32
    )
    m = jnp.max(logits, axis=-1, keepdims=True)
    e = jnp.exp(logits - m)
    s = jnp.sum(e, axis=-1, keepdims=True)
    probs = e / s
    probs_ref[...] = probs

    work = probs
    iota = jax.lax.broadcasted_iota(jnp.int32, probs.shape, 1).astype(
        jnp.float32
    )
    vals = []
    idxs = []
    for _ in range(TOP_K):
        v = jnp.max(work, axis=-1, keepdims=True)
        i = jnp.min(
            jnp.where(work == v, iota, float(N_EXPERTS)),
            axis=-1,
            keepdims=True,
        )
        vals.append(v)
        idxs.append(i)
        work = jnp.where(iota == i, -jnp.inf, work)
    top_vals = jnp.concatenate(vals, axis=-1)
    top_idx = jnp.concatenate(idxs, axis=-1)
    top_vals = top_vals / jnp.sum(top_vals, axis=-1, keepdims=True)
    vals_ref[...] = top_vals
    idx_ref[...] = top_idx.astype(jnp.int32)


@jax.jit
def kernel(x, W):
    n_tiles = TOKENS // TILE
    probs, top_vals, top_idx = pl.pallas_call(
        _gate_kernel,
        grid=(n_tiles,),
        in_specs=[
            pl.BlockSpec((TILE, HIDDEN), lambda i: (i, 0)),
            pl.BlockSpec((N_EXPERTS, HIDDEN), lambda i: (0, 0)),
        ],
        out_specs=[
            pl.BlockSpec((TILE, N_EXPERTS), lambda i: (i, 0)),
            pl.BlockSpec((TILE, TOP_K), lambda i: (i, 0)),
            pl.BlockSpec((TILE, TOP_K), lambda i: (i, 0)),
        ],
        out_shape=[
            jax.ShapeDtypeStruct((TOKENS, N_EXPERTS), jnp.float32),
            jax.ShapeDtypeStruct((TOKENS, TOP_K), jnp.float32),
            jax.ShapeDtypeStruct((TOKENS, TOP_K), jnp.int32),
        ],
        compiler_params=pltpu.CompilerParams(
            dimension_semantics=("parallel",),
        ),
    )(x, W)
    return (probs, top_vals, top_idx)
